# baseline (speedup 1.0000x reference)
import jax
import jax.numpy as jnp
from jax import lax
from jax.experimental import pallas as pl
from jax.experimental.pallas import tpu as pltpu
from jax.experimental.pallas import tpu_sc as plsc

_B = 16384
_D = 64
_N = 1000000
_NCOL = (_N + 127) // 128
_NC, _NS, _L = 2, 16, 16
_NW = _NC * _NS
_BPW = _B // _NW
_CPW = 256
_CH = 128
_NCH = _BPW // _CH
_SW = 128
_DUMP = _B


def _extract_body(uidx_hbm, iidx_hbm, eu_hbm, ei_hbm, ustag_hbm, istag_hbm,
                  idx_full, my_i, my_b, win, rows_buf, idx_w, tmp_l,
                  wsem, ssem):
    wid = lax.axis_index("s") * _NC + lax.axis_index("c")
    iota16 = lax.iota(jnp.int32, _L)

    for idx_hbm, tbl_hbm, stag_hbm in ((uidx_hbm, eu_hbm, ustag_hbm),
                                       (iidx_hbm, ei_hbm, istag_hbm)):
        pltpu.sync_copy(idx_hbm, idx_full)

        def scan_body(t, off):
            vec = idx_full[pl.ds(t * _L, _L)]
            m = lax.shift_right_logical(vec, 15) == wid
            pc = plsc.all_reduce_population_count(m)[0]
            plsc.store_compressed(my_i.at[pl.ds(off, _L)], vec, mask=m)
            plsc.store_compressed(my_b.at[pl.ds(off, _L)], t * _L + iota16, mask=m)
            return off + pc

        cnt_my = lax.fori_loop(0, _B // _L, scan_body, 0)
        my_i[pl.ds(cnt_my, _L)] = jnp.full((_L,), jnp.int32(0x7FFFFFFF))
        nvreg = lax.shift_right_logical(cnt_my + 15, 4)

        def issue(col, slot):
            cs = jnp.minimum(col, _NCOL - 1) * 128
            pltpu.async_copy(tbl_hbm.at[:, pl.ds(cs, 128)], win.at[slot], wsem)

        for t in range(3):
            issue(wid * _CPW + t, t)

        def col_body(c_rel, nscat):
            slot = lax.rem(c_rel, 4)

            @pl.when(c_rel < _CPW - 3)
            def _():
                issue(wid * _CPW + c_rel + 3, lax.rem(c_rel + 3, 4))

            pltpu.make_async_copy(tbl_hbm.at[:, pl.ds(0, 128)],
                                  win.at[slot], wsem).wait()
            c_abs = jnp.minimum(wid * _CPW + c_rel, _NCOL - 1)

            def vreg_body(j, ns):
                ivec = my_i[pl.ds(j * _L, _L)]
                m = lax.shift_right_logical(ivec, 7) == c_abs
                pc = plsc.all_reduce_population_count(m)[0]

                @pl.when(pc > 0)
                def _():
                    sslot = lax.rem(ns, 4)

                    @pl.when(ns >= 4)
                    def _():
                        pltpu.make_async_copy(
                            ustag_hbm.at[pl.ds(0, _L)],
                            rows_buf.at[sslot], ssem).wait()

                    idx_w.at[sslot][...] = jnp.full((_L,), jnp.int32(_DUMP))
                    bvec = my_b[pl.ds(j * _L, _L)]
                    plsc.store_compressed(idx_w.at[sslot], bvec, mask=m)
                    tmp_l[...] = jnp.full((_L,), jnp.int32(0))
                    plsc.store_compressed(tmp_l.at[pl.ds(0, _L)], ivec & 127, mask=m)
                    lv = tmp_l[...]
                    for k in range(_L):
                        @pl.when(k < pc)
                        def _():
                            lsplat = jnp.full((_L,), lv[k], jnp.int32)
                            for kk in range(_D // _L):
                                vals = plsc.load_gather(
                                    win.at[slot], [kk * _L + iota16, lsplat])
                                rows_buf.at[sslot].at[k][pl.ds(kk * _L, _L)] = vals
                    pltpu.async_copy(rows_buf.at[sslot],
                                     stag_hbm.at[idx_w.at[sslot]], ssem)

                return ns + jnp.where(pc > 0, 1, 0)

            return lax.fori_loop(0, nvreg, vreg_body, nscat)

        nscat = lax.fori_loop(0, _CPW, col_body, 0)

        def drain_body(t, carry):
            pltpu.make_async_copy(ustag_hbm.at[pl.ds(0, _L)],
                                  rows_buf.at[lax.rem(t, 4)], ssem).wait()
            return carry

        lax.fori_loop(0, jnp.minimum(nscat, 4), drain_body, 0)


def _dot_body(iidx_hbm, ustag_hbm, istag_hbm, bias_hbm, out_hbm,
              iidx_v, urows_v, irows_v, bias_v, out_v, sem):
    wid = lax.axis_index("s") * _NC + lax.axis_index("c")
    base = wid * _BPW
    pltpu.sync_copy(iidx_hbm.at[pl.ds(base, _BPW)], iidx_v)
    pltpu.sync_copy(ustag_hbm.at[pl.ds(base, _BPW), pl.ds(0, _D)], urows_v)
    pltpu.sync_copy(istag_hbm.at[pl.ds(base, _BPW), pl.ds(0, _D)], irows_v)
    copies = []
    for j in range(_NCH):
        s = pl.ds(j * _CH, _CH)
        copies.append(pltpu.async_copy(bias_hbm.at[iidx_v.at[s]], bias_v.at[s], sem))
    for c in copies:
        c.wait()

    iota16 = lax.iota(jnp.int32, _L)

    def group_body(g, carry):
        rows = g * _L + iota16
        bias16 = bias_v[pl.ds(g * _L, _L)]

        def d_body(d, acc):
            cols = jnp.full((_L,), d, jnp.int32)
            u = plsc.load_gather(urows_v, [rows, cols])
            it = plsc.load_gather(irows_v, [rows, cols])
            return acc + u * it

        acc = lax.fori_loop(0, _D, d_body, bias16)
        out_v[pl.ds(g * _L, _L)] = acc
        return carry

    lax.fori_loop(0, _BPW // _L, group_body, 0)
    pltpu.sync_copy(out_v, out_hbm.at[pl.ds(base, _BPW)])


def kernel(user_indices, item_indices, embedding_user, embedding_item, bias_item):
    ui = user_indices.astype(jnp.int32)
    ii = item_indices.astype(jnp.int32)
    mesh = plsc.VectorSubcoreMesh(core_axis_name="c", subcore_axis_name="s")

    ustag, istag = pl.kernel(
        _extract_body,
        out_type=(jax.ShapeDtypeStruct((_B + 1, _SW), jnp.float32),
                  jax.ShapeDtypeStruct((_B + 1, _SW), jnp.float32)),
        mesh=mesh,
        compiler_params=pltpu.CompilerParams(needs_layout_passes=False),
        scratch_types=[
            pltpu.VMEM((_B,), jnp.int32),
            pltpu.VMEM((_B + _L,), jnp.int32),
            pltpu.VMEM((_B + _L,), jnp.int32),
            pltpu.VMEM((4, _D, 128), jnp.float32),
            pltpu.VMEM((4, _L, _SW), jnp.float32),
            pltpu.VMEM((4, _L), jnp.int32),
            pltpu.VMEM((_L,), jnp.int32),
            pltpu.SemaphoreType.DMA,
            pltpu.SemaphoreType.DMA,
        ],
    )(ui, ii, embedding_user.T, embedding_item.T)

    out = pl.kernel(
        _dot_body,
        out_type=jax.ShapeDtypeStruct((_B,), jnp.float32),
        mesh=mesh,
        compiler_params=pltpu.CompilerParams(
            needs_layout_passes=False, use_tc_tiling_on_sc=False
        ),
        scratch_types=[
            pltpu.VMEM((_BPW,), jnp.int32),
            pltpu.VMEM((_BPW, _D), jnp.float32),
            pltpu.VMEM((_BPW, _D), jnp.float32),
            pltpu.VMEM((_BPW,), jnp.float32),
            pltpu.VMEM((_BPW,), jnp.float32),
            pltpu.SemaphoreType.DMA,
        ],
    )(ii, ustag, istag, bias_item.reshape(-1))
    return out


# speedup vs baseline: 40.3653x; 40.3653x over previous
import jax
import jax.numpy as jnp
from jax import lax
from jax.experimental import pallas as pl
from jax.experimental.pallas import tpu as pltpu
from jax.experimental.pallas import tpu_sc as plsc

_B = 16384
_D = 64
_N = 1000000
_NC, _NS, _L = 2, 16, 16
_NW = _NC * _NS
_BPW = _B // _NW
_CH = 128
_NCH = _BPW // _CH


def _bias_body(iidx_hbm, bias_hbm, out_hbm, iidx_v, bias_v, sem):
    wid = lax.axis_index("s") * _NC + lax.axis_index("c")
    base = wid * _BPW
    pltpu.sync_copy(iidx_hbm.at[pl.ds(base, _BPW)], iidx_v)
    copies = []
    for j in range(_NCH):
        s = pl.ds(j * _CH, _CH)
        copies.append(pltpu.async_copy(bias_hbm.at[iidx_v.at[s]], bias_v.at[s], sem))
    for c in copies:
        c.wait()
    pltpu.sync_copy(bias_v, out_hbm.at[pl.ds(base, _BPW)])


def _main_body(uidx_hbm, iidx_hbm, eu_hbm, ei_hbm, b16_hbm, out_hbm,
               uidx_s, iidx_s, u_win, i_win, bias_v, out_v, usem, isem):
    wid = lax.axis_index("s") * _NC + lax.axis_index("c")
    base = wid * _BPW
    pltpu.sync_copy(uidx_hbm.at[pl.ds(base, _BPW)], uidx_s)
    pltpu.sync_copy(iidx_hbm.at[pl.ds(base, _BPW)], iidx_s)
    pltpu.sync_copy(b16_hbm.at[pl.ds(base, _BPW)], bias_v)

    iota16 = lax.iota(jnp.int32, _L)

    def issue(nu, ni, slot):
        cu = lax.shift_right_logical(nu, 7) * 128
        ci = lax.shift_right_logical(ni, 7) * 128
        pltpu.async_copy(eu_hbm.at[:, pl.ds(cu, 128)], u_win.at[slot], usem)
        pltpu.async_copy(ei_hbm.at[:, pl.ds(ci, 128)], i_win.at[slot], isem)

    _NBUF = 4
    _AHEAD = _NBUF - 1
    uv0 = uidx_s[pl.ds(0, _L)]
    iv0 = iidx_s[pl.ds(0, _L)]
    for t in range(_AHEAD):
        issue(uv0[t], iv0[t], t)

    def group_body(g, carry):
        uvec = uidx_s[pl.ds(g * _L, _L)]
        ivec = iidx_s[pl.ds(g * _L, _L)]
        gn = jnp.minimum(g + 1, _BPW // _L - 1)
        uvec_n = uidx_s[pl.ds(gn * _L, _L)]
        ivec_n = iidx_s[pl.ds(gn * _L, _L)]
        res = bias_v[pl.ds(g * _L, _L)]
        for k in range(_L):
            slot = k & (_NBUF - 1)
            nslot = (k + _AHEAD) & (_NBUF - 1)
            if k + _AHEAD < _L:
                issue(uvec[k + _AHEAD], ivec[k + _AHEAD], nslot)
            else:
                @pl.when(g < _BPW // _L - 1)
                def _():
                    issue(uvec_n[k + _AHEAD - _L], ivec_n[k + _AHEAD - _L], nslot)

            pltpu.make_async_copy(eu_hbm.at[:, pl.ds(0, 128)], u_win.at[slot], usem).wait()
            pltpu.make_async_copy(ei_hbm.at[:, pl.ds(0, 128)], i_win.at[slot], isem).wait()

            lu = jnp.full((_L,), uvec[k] & 127, jnp.int32)
            li = jnp.full((_L,), ivec[k] & 127, jnp.int32)
            acc = jnp.zeros((_L,), jnp.float32)
            for kk in range(_D // _L):
                rows = kk * _L + iota16
                u = plsc.load_gather(u_win.at[slot], [rows, lu])
                v = plsc.load_gather(i_win.at[slot], [rows, li])
                acc = acc + u * v
            s = jnp.sum(acc)
            res = jnp.where(iota16 == k, res + s, res)
        out_v[pl.ds(g * _L, _L)] = res
        return carry

    lax.fori_loop(0, _BPW // _L, group_body, 0)
    pltpu.sync_copy(out_v, out_hbm.at[pl.ds(base, _BPW)])


def kernel(user_indices, item_indices, embedding_user, embedding_item, bias_item):
    ui = user_indices.astype(jnp.int32)
    ii = item_indices.astype(jnp.int32)
    mesh = plsc.VectorSubcoreMesh(core_axis_name="c", subcore_axis_name="s")

    bias16 = pl.kernel(
        _bias_body,
        out_type=jax.ShapeDtypeStruct((_B,), jnp.float32),
        mesh=mesh,
        compiler_params=pltpu.CompilerParams(
            needs_layout_passes=False, use_tc_tiling_on_sc=False
        ),
        scratch_types=[
            pltpu.VMEM((_BPW,), jnp.int32),
            pltpu.VMEM((_BPW,), jnp.float32),
            pltpu.SemaphoreType.DMA,
        ],
    )(ii, bias_item.reshape(-1))

    out = pl.kernel(
        _main_body,
        out_type=jax.ShapeDtypeStruct((_B,), jnp.float32),
        mesh=mesh,
        compiler_params=pltpu.CompilerParams(needs_layout_passes=False),
        scratch_types=[
            pltpu.VMEM((_BPW,), jnp.int32),
            pltpu.VMEM((_BPW,), jnp.int32),
            pltpu.VMEM((4, _D, 128), jnp.float32),
            pltpu.VMEM((4, _D, 128), jnp.float32),
            pltpu.VMEM((_BPW,), jnp.float32),
            pltpu.VMEM((_BPW,), jnp.float32),
            pltpu.SemaphoreType.DMA,
            pltpu.SemaphoreType.DMA,
        ],
    )(ui, ii, embedding_user.T, embedding_item.T, bias16)
    return out
